# Initial kernel scaffold; baseline (speedup 1.0000x reference)
import jax
import jax.numpy as jnp
from jax.experimental import pallas as pl


def kernel(x, edge_index, W1, b1, W2, b2):
    raise NotImplementedError("write your pallas kernel here")


# speedup vs baseline: 10.8474x; 10.8474x over previous
import functools

import jax
import jax.numpy as jnp
from jax import lax
from jax.experimental import pallas as pl
from jax.experimental.pallas import tpu as pltpu
from jax.experimental.pallas import tpu_sc as plsc

N = 10000
D = 128
E = 320000

NC = 2
NS = 16
NW = NC * NS

C = 128
EP = 10112
EPAD = EP * NW
NCH = EP // C

NP = 10240
ZR = NP // NS

_mesh = plsc.VectorSubcoreMesh(core_axis_name="c", subcore_axis_name="s")
_sc_params = pltpu.CompilerParams(use_tc_tiling_on_sc=False)


def _fill(ref, nrows, ncols, value):
    v = jnp.full((16,), value, jnp.float32)

    def body(i, carry):
        for c in range(ncols // 16):
            ref[i, pl.ds(c * 16, 16)] = v
        return carry

    lax.fori_loop(0, nrows, body, 0)


@functools.partial(
    pl.kernel,
    out_type=jax.ShapeDtypeStruct((NC, NP, 16), jnp.float32),
    mesh=_mesh,
    scratch_types=[
        pltpu.VMEM((1, C), jnp.int32),
        pltpu.VMEM((C, 16), jnp.float32),
        pltpu.VMEM_SHARED((NP, 16), jnp.float32),
    ],
    compiler_params=_sc_params,
)
def _sc_deg(dst_hbm, out_hbm, didx, ones_v, acc):
    cid = lax.axis_index("c")
    sid = lax.axis_index("s")

    _fill(ones_v, C, 16, 0.0)
    zbase = sid * ZR
    for k in range(ZR // C):
        pltpu.sync_copy(ones_v, acc.at[pl.ds(zbase + k * C, C)])
    _fill(ones_v, C, 16, 1.0)
    plsc.subcore_barrier()

    ebase = (cid * NS + sid) * EP

    def chunk(i, carry):
        b = ebase + i * C
        pltpu.sync_copy(dst_hbm.at[pl.ds(b, C)], didx.at[0])
        pltpu.sync_copy(ones_v, acc.at[didx.at[0]], add=True)
        return carry

    lax.fori_loop(0, NCH, chunk, 0)
    plsc.subcore_barrier()

    pltpu.sync_copy(acc.at[pl.ds(zbase, ZR)],
                    out_hbm.at[cid, pl.ds(zbase, ZR)])


@functools.partial(
    pl.kernel,
    out_type=jax.ShapeDtypeStruct((NC, NP, D), jnp.float32),
    mesh=_mesh,
    scratch_types=[
        pltpu.VMEM((C,), jnp.int32),
        pltpu.VMEM((1, C), jnp.int32),
        pltpu.VMEM((C, D), jnp.float32),
        pltpu.VMEM_SHARED((NP, D), jnp.float32),
        pltpu.SemaphoreType.DMA,
    ],
    compiler_params=_sc_params,
)
def _sc_agg(g_hbm, src_hbm, dst_hbm, out_hbm, sidx, didx, rows, acc, sem):
    cid = lax.axis_index("c")
    sid = lax.axis_index("s")

    _fill(rows, C, D, 0.0)
    zbase = sid * ZR
    for k in range(ZR // C):
        pltpu.sync_copy(rows, acc.at[pl.ds(zbase + k * C, C)])
    plsc.subcore_barrier()

    ebase = (cid * NS + sid) * EP

    def chunk(i, carry):
        b = ebase + i * C
        pltpu.sync_copy(src_hbm.at[pl.ds(b, C)], sidx)
        pltpu.sync_copy(dst_hbm.at[pl.ds(b, C)], didx.at[0])
        pltpu.async_copy(g_hbm.at[sidx], rows, sem).wait()
        pltpu.sync_copy(rows, acc.at[didx.at[0]], add=True)
        return carry

    lax.fori_loop(0, NCH, chunk, 0)
    plsc.subcore_barrier()

    pltpu.sync_copy(acc.at[pl.ds(zbase, ZR)],
                    out_hbm.at[cid, pl.ds(zbase, ZR)])


_RB = 1000


def _dis_of(dref):
    deg = dref[0, :, 0:1] + dref[1, :, 0:1] + 1.0
    return lax.rsqrt(deg)


def _tc1_body(x_ref, w_ref, d_ref, o_ref):
    dis = _dis_of(d_ref)
    h = jnp.dot(x_ref[...], w_ref[...], preferred_element_type=jnp.float32)
    o_ref[...] = h * dis


def _tc2_body(a_ref, g_ref, d_ref, w_ref, b_ref, o_ref):
    dis = _dis_of(d_ref)
    s = a_ref[0] + a_ref[1] + g_ref[...]
    r = jnp.maximum(s * dis + b_ref[...], 0.0)
    o_ref[...] = jnp.dot(r, w_ref[...], preferred_element_type=jnp.float32) * dis


def _tc3_body(a_ref, g_ref, d_ref, b_ref, o_ref):
    dis = _dis_of(d_ref)
    s = a_ref[0] + a_ref[1] + g_ref[...]
    o_ref[...] = s * dis + b_ref[...]


def _row_spec(i):
    return (i, 0)


_spec_rows = pl.BlockSpec((_RB, D), _row_spec)
_spec_acc = pl.BlockSpec((NC, _RB, D), lambda i: (0, i, 0))
_spec_deg = pl.BlockSpec((NC, _RB, 16), lambda i: (0, i, 0))
_spec_w = pl.BlockSpec((D, D), lambda i: (0, 0))
_spec_b = pl.BlockSpec((1, D), lambda i: (0, 0))

_GRID = (N // _RB,)
_out_rows = jax.ShapeDtypeStruct((N, D), jnp.float32)

_tc1 = pl.pallas_call(
    _tc1_body, grid=_GRID,
    in_specs=[_spec_rows, _spec_w, _spec_deg],
    out_specs=_spec_rows, out_shape=_out_rows)

_tc2 = pl.pallas_call(
    _tc2_body, grid=_GRID,
    in_specs=[_spec_acc, _spec_rows, _spec_deg, _spec_w, _spec_b],
    out_specs=_spec_rows, out_shape=_out_rows)

_tc3 = pl.pallas_call(
    _tc3_body, grid=_GRID,
    in_specs=[_spec_acc, _spec_rows, _spec_deg, _spec_b],
    out_specs=_spec_rows, out_shape=_out_rows)


def _jax_deg(dst):
    parts = []
    for c in range(NC):
        seg = dst[c * NS * EP:(c + 1) * NS * EP]
        dcount = jnp.zeros((NP,), jnp.float32).at[seg].add(1.0)
        parts.append(jnp.broadcast_to(dcount[:, None], (NP, 16)))
    return jnp.stack(parts)


def _jax_agg(g, src, dst):
    parts = []
    for c in range(NC):
        s = src[c * NS * EP:(c + 1) * NS * EP]
        t = dst[c * NS * EP:(c + 1) * NS * EP]
        parts.append(jnp.zeros((NP, D), jnp.float32).at[t].add(g[s]))
    return jnp.stack(parts)


def kernel(x, edge_index, W1, b1, W2, b2):
    ei = edge_index.astype(jnp.int32)
    pad = EPAD - E
    src = jnp.concatenate([ei[0], jnp.zeros((pad,), jnp.int32)])
    dst = jnp.concatenate([ei[1], jnp.full((pad,), N, jnp.int32)])
    b1r = b1.reshape(1, D)
    b2r = b2.reshape(1, D)

    degp = _sc_deg(dst)
    g1 = _tc1(x, W1, degp)
    acc1 = _sc_agg(g1, src, dst)
    g2 = _tc2(acc1, g1, degp, W2, b1r)
    acc2 = _sc_agg(g2, src, dst)
    return _tc3(acc2, g2, degp, b2r)
